# baseline (speedup 1.0000x reference)
import functools

import jax
import jax.numpy as jnp
from jax import lax
from jax.experimental import pallas as pl
from jax.experimental.pallas import tpu as pltpu
from jax.experimental.pallas import tpu_sc as plsc

_NREL = 474
_B = 16384
_NW = 32
_BPW = _B // _NW
_L = 16


_mesh = plsc.VectorSubcoreMesh(core_axis_name="c", subcore_axis_name="s")


@functools.partial(
    pl.kernel,
    out_type=jax.ShapeDtypeStruct((2, _B), jnp.float32),
    mesh=_mesh,
    scratch_types=[
        pltpu.VMEM((3, _BPW), jnp.int32),
        pltpu.VMEM((_NREL,), jnp.float32),
        pltpu.VMEM((_NREL,), jnp.float32),
        pltpu.VMEM((2, _BPW), jnp.float32),
        pltpu.SemaphoreType.DMA,
    ],
    compiler_params=pltpu.CompilerParams(needs_layout_passes=False),
)
def _prior_weight_sc(pos_hbm, cur_hbm, kra_hbm, out_hbm,
                     pos_v, cur_v, kra_v, out_v, sem):
    wid = lax.axis_index("s") * 2 + lax.axis_index("c")
    base = wid * _BPW

    cp_pos = pltpu.async_copy(pos_hbm.at[:, pl.ds(base, _BPW)], pos_v, sem)
    cp_cur = pltpu.async_copy(cur_hbm, cur_v, sem)
    cp_kra = pltpu.async_copy(kra_hbm, kra_v, sem)
    cp_pos.wait()
    cp_cur.wait()
    cp_kra.wait()

    iota = lax.broadcasted_iota(jnp.int32, (_L,), 0)
    zeros = iota * 0
    ones = zeros + 1

    @plsc.parallel_loop(0, _BPW // _L, unroll=8)
    def body(j):
        cols = iota + j * _L
        ridx = pos_v[1, pl.ds(j * _L, _L)]
        c = plsc.load_gather(cur_v, [ridx])
        k = plsc.load_gather(kra_v, [ridx])
        t1 = (jnp.where(c < 0.5, 0.7, 0.3)
              + jnp.where(k > 0.5, 0.7, 0.3)) * 0.5
        t0 = 1.0 - t1
        plsc.store_scatter(out_v, [zeros, cols], t0)
        plsc.store_scatter(out_v, [ones, cols], t1)

    pltpu.sync_copy(out_v, out_hbm.at[:, pl.ds(base, _BPW)])


def kernel(positive_sample, negative_sample, cur, kra):
    del negative_sample
    pos_t = positive_sample.astype(jnp.int32).T
    out_t = _prior_weight_sc(pos_t, cur, kra)
    return out_t.T[:, None, :]


# speedup vs baseline: 1.0081x; 1.0081x over previous
import functools

import jax
import jax.numpy as jnp
from jax import lax
from jax.experimental import pallas as pl
from jax.experimental.pallas import tpu as pltpu
from jax.experimental.pallas import tpu_sc as plsc

_NREL = 474
_B = 16384
_NW = 32
_BPW = _B // _NW
_L = 16


_mesh = plsc.VectorSubcoreMesh(core_axis_name="c", subcore_axis_name="s")


@functools.partial(
    pl.kernel,
    out_type=jax.ShapeDtypeStruct((2, _B), jnp.float32),
    mesh=_mesh,
    scratch_types=[
        pltpu.VMEM((3, _BPW), jnp.int32),
        pltpu.VMEM((_NREL,), jnp.float32),
        pltpu.VMEM((_NREL,), jnp.float32),
        pltpu.VMEM((2, _BPW), jnp.float32),
        pltpu.SemaphoreType.DMA,
    ],
    compiler_params=pltpu.CompilerParams(needs_layout_passes=False),
)
def _prior_weight_sc(pos_hbm, cur_hbm, kra_hbm, out_hbm,
                     pos_v, cur_v, kra_v, out_v, sem):
    wid = lax.axis_index("s") * 2 + lax.axis_index("c")
    base = wid * _BPW

    cp_pos = pltpu.async_copy(pos_hbm.at[:, pl.ds(base, _BPW)], pos_v, sem)
    cp_cur = pltpu.async_copy(cur_hbm, cur_v, sem)
    cp_kra = pltpu.async_copy(kra_hbm, kra_v, sem)
    cp_pos.wait()
    cp_cur.wait()
    cp_kra.wait()

    iota = lax.broadcasted_iota(jnp.int32, (_L,), 0)
    zeros = iota * 0
    ones = zeros + 1

    @plsc.parallel_loop(0, _BPW // _L, unroll=4)
    def body(j):
        cols = iota + j * _L
        ridx = pos_v[1, pl.ds(j * _L, _L)]
        c = plsc.load_gather(cur_v, [ridx])
        k = plsc.load_gather(kra_v, [ridx])
        t1 = (jnp.where(c < 0.5, 0.7, 0.3)
              + jnp.where(k > 0.5, 0.7, 0.3)) * 0.5
        t0 = 1.0 - t1
        plsc.store_scatter(out_v, [zeros, cols], t0)
        plsc.store_scatter(out_v, [ones, cols], t1)

    pltpu.sync_copy(out_v, out_hbm.at[:, pl.ds(base, _BPW)])


def kernel(positive_sample, negative_sample, cur, kra):
    del negative_sample
    pos_t = positive_sample.astype(jnp.int32).T
    out_t = _prior_weight_sc(pos_t, cur, kra)
    return out_t.T[:, None, :]
